# baseline (device time: 17793 ns/iter reference)
import jax
import jax.numpy as jnp
from jax import lax
from jax.experimental import pallas as pl
from jax.experimental.pallas import tpu as pltpu

N_DEV = 4
EXPERTS_PER_DEV = 2
NC = 8
LAG = 2


def kernel(x, router_W, route_idx, expert_W):
    del router_W
    n_tok, _ = x.shape
    _, _, h = expert_W.shape
    rpc = n_tok // NC

    def body(x_ref, idx_ref, w_ref, out_ref,
             send1, recv1, send2, recv2, ss1, rs1, ss2, rs2):
        my_pos = lax.axis_index("i")
        pA = jnp.bitwise_xor(my_pos, 1)
        pB = 3 - my_pos

        barrier_sem = pltpu.get_barrier_semaphore()
        for nbr in (pA, pB):
            pl.semaphore_signal(
                barrier_sem, inc=1,
                device_id=(nbr,), device_id_type=pl.DeviceIdType.MESH,
            )
        pl.semaphore_wait(barrier_sem, 2)

        e0 = my_pos * EXPERTS_PER_DEV
        w0 = w_ref[0, :, :].astype(jnp.bfloat16)
        w1 = w_ref[1, :, :].astype(jnp.bfloat16)

        rdma1 = []
        for c in range(NC):
            rs = slice(c * rpc, (c + 1) * rpc)
            idx_c = idx_ref[rs, :]
            x_c = x_ref[rs, :]
            x0 = jnp.where(idx_c == e0, x_c, 0.0).astype(jnp.bfloat16)
            x1 = jnp.where(idx_c == e0 + 1, x_c, 0.0).astype(jnp.bfloat16)
            send1[c, :, :] = (
                lax.dot(x0, w0, preferred_element_type=jnp.float32)
                + lax.dot(x1, w1, preferred_element_type=jnp.float32)
            ).astype(jnp.bfloat16)
            d = pltpu.make_async_remote_copy(
                src_ref=send1.at[c], dst_ref=recv1.at[c],
                send_sem=ss1.at[c], recv_sem=rs1.at[c],
                device_id=(pA,), device_id_type=pl.DeviceIdType.MESH,
            )
            d.start()
            rdma1.append(d)

        def finish(c):
            rdma2[c].wait_recv()
            out_ref[c * rpc:(c + 1) * rpc, :] = (
                send2[c, :, :] + recv2[c, :, :]
            ).astype(jnp.float32)

        rdma2 = []
        for c in range(NC):
            rdma1[c].wait_recv()
            send2[c, :, :] = send1[c, :, :] + recv1[c, :, :]
            d = pltpu.make_async_remote_copy(
                src_ref=send2.at[c], dst_ref=recv2.at[c],
                send_sem=ss2.at[c], recv_sem=rs2.at[c],
                device_id=(pB,), device_id_type=pl.DeviceIdType.MESH,
            )
            d.start()
            rdma2.append(d)
            if c >= LAG:
                finish(c - LAG)
        for c in range(NC - LAG, NC):
            finish(c)

        for c in range(NC):
            rdma1[c].wait_send()
            rdma2[c].wait_send()

    return pl.pallas_call(
        body,
        out_shape=jax.ShapeDtypeStruct((n_tok, h), jnp.float32),
        in_specs=[pl.BlockSpec(memory_space=pltpu.VMEM)] * 3,
        out_specs=pl.BlockSpec(memory_space=pltpu.VMEM),
        scratch_shapes=[
            pltpu.VMEM((NC, rpc, h), jnp.bfloat16),
            pltpu.VMEM((NC, rpc, h), jnp.bfloat16),
            pltpu.VMEM((NC, rpc, h), jnp.bfloat16),
            pltpu.VMEM((NC, rpc, h), jnp.bfloat16),
            pltpu.SemaphoreType.DMA((NC,)),
            pltpu.SemaphoreType.DMA((NC,)),
            pltpu.SemaphoreType.DMA((NC,)),
            pltpu.SemaphoreType.DMA((NC,)),
        ],
        compiler_params=pltpu.CompilerParams(collective_id=0),
    )(x, route_idx, expert_W)


# device time: 17336 ns/iter; 1.0264x vs baseline; 1.0264x over previous
import jax
import jax.numpy as jnp
from jax import lax
from jax.experimental import pallas as pl
from jax.experimental.pallas import tpu as pltpu

N_DEV = 4
EXPERTS_PER_DEV = 2
NC = 8
LAG = 2


def kernel(x, router_W, route_idx, expert_W):
    del router_W
    n_tok, _ = x.shape
    _, _, h = expert_W.shape
    rpc = n_tok // NC

    def body(x_ref, idx_ref, w_ref, out_ref,
             send1, recv1, send2, recv2, ss1, rs1, ss2, rs2):
        my_pos = lax.axis_index("i")
        pA = jnp.bitwise_xor(my_pos, 1)
        pB = 3 - my_pos

        barrier_sem = pltpu.get_barrier_semaphore()
        for nbr in (pA, pB):
            pl.semaphore_signal(
                barrier_sem, inc=1,
                device_id=(nbr,), device_id_type=pl.DeviceIdType.MESH,
            )

        e0 = my_pos * EXPERTS_PER_DEV
        w0 = w_ref[0, :, :].astype(jnp.bfloat16)
        w1 = w_ref[1, :, :].astype(jnp.bfloat16)

        rdma1 = []
        for c in range(NC):
            rs = slice(c * rpc, (c + 1) * rpc)
            idx_c = idx_ref[rs, :]
            x_c = x_ref[rs, :]
            x0 = jnp.where(idx_c == e0, x_c, 0.0).astype(jnp.bfloat16)
            x1 = jnp.where(idx_c == e0 + 1, x_c, 0.0).astype(jnp.bfloat16)
            send1[c, :, :] = (
                lax.dot(x0, w0, preferred_element_type=jnp.float32)
                + lax.dot(x1, w1, preferred_element_type=jnp.float32)
            ).astype(jnp.bfloat16)
            if c == 0:
                pl.semaphore_wait(barrier_sem, 2)
            d = pltpu.make_async_remote_copy(
                src_ref=send1.at[c], dst_ref=recv1.at[c],
                send_sem=ss1.at[c], recv_sem=rs1.at[c],
                device_id=(pA,), device_id_type=pl.DeviceIdType.MESH,
            )
            d.start()
            rdma1.append(d)

        def finish(c):
            rdma2[c].wait_recv()
            out_ref[c * rpc:(c + 1) * rpc, :] = send2[c, :, :] + recv2[c, :, :]

        rdma2 = []
        for c in range(NC):
            rdma1[c].wait_recv()
            send2[c, :, :] = send1[c, :, :] + recv1[c, :, :]
            d = pltpu.make_async_remote_copy(
                src_ref=send2.at[c], dst_ref=recv2.at[c],
                send_sem=ss2.at[c], recv_sem=rs2.at[c],
                device_id=(pB,), device_id_type=pl.DeviceIdType.MESH,
            )
            d.start()
            rdma2.append(d)
            if c >= LAG:
                finish(c - LAG)
        for c in range(NC - LAG, NC):
            finish(c)

        for c in range(NC):
            rdma1[c].wait_send()
            rdma2[c].wait_send()

    return pl.pallas_call(
        body,
        out_shape=jax.ShapeDtypeStruct((n_tok, h), jnp.bfloat16),
        in_specs=[pl.BlockSpec(memory_space=pltpu.VMEM)] * 3,
        out_specs=pl.BlockSpec(memory_space=pltpu.VMEM),
        scratch_shapes=[
            pltpu.VMEM((NC, rpc, h), jnp.bfloat16),
            pltpu.VMEM((NC, rpc, h), jnp.bfloat16),
            pltpu.VMEM((NC, rpc, h), jnp.bfloat16),
            pltpu.VMEM((NC, rpc, h), jnp.bfloat16),
            pltpu.SemaphoreType.DMA((NC,)),
            pltpu.SemaphoreType.DMA((NC,)),
            pltpu.SemaphoreType.DMA((NC,)),
            pltpu.SemaphoreType.DMA((NC,)),
        ],
        compiler_params=pltpu.CompilerParams(collective_id=0),
    )(x, route_idx, expert_W)


# device time: 16490 ns/iter; 1.0790x vs baseline; 1.0513x over previous
import jax
import jax.numpy as jnp
from jax import lax
from jax.experimental import pallas as pl
from jax.experimental.pallas import tpu as pltpu

N_DEV = 4
EXPERTS_PER_DEV = 2
NC = 4
LAG = 2


def kernel(x, router_W, route_idx, expert_W):
    del router_W
    n_tok, d = x.shape
    _, _, h = expert_W.shape
    rpc = n_tok // NC

    def body(x_hbm, idx_ref, w_hbm, out_hbm,
             x_vmem, w_vmem, outbuf, send1, recv1, send2, recv2,
             in_sems, out_sems, ss1, rs1, ss2, rs2):
        my_pos = lax.axis_index("i")
        pA = jnp.bitwise_xor(my_pos, 1)
        pB = 3 - my_pos

        cp_x = pltpu.make_async_copy(x_hbm, x_vmem, in_sems.at[0])
        cp_w = pltpu.make_async_copy(w_hbm, w_vmem, in_sems.at[1])
        cp_x.start()
        cp_w.start()

        barrier_sem = pltpu.get_barrier_semaphore()
        for nbr in (pA, pB):
            pl.semaphore_signal(
                barrier_sem, inc=1,
                device_id=(nbr,), device_id_type=pl.DeviceIdType.MESH,
            )

        cp_w.wait()
        cp_x.wait()
        e0 = my_pos * EXPERTS_PER_DEV
        w0 = w_vmem[0, :, :].astype(jnp.bfloat16)
        w1 = w_vmem[1, :, :].astype(jnp.bfloat16)

        rdma1 = []
        for c in range(NC):
            rs = slice(c * rpc, (c + 1) * rpc)
            idx_c = idx_ref[rs, :]
            x_c = x_vmem[rs, :]
            x0 = jnp.where(idx_c == e0, x_c, 0.0).astype(jnp.bfloat16)
            x1 = jnp.where(idx_c == e0 + 1, x_c, 0.0).astype(jnp.bfloat16)
            send1[c, :, :] = (
                lax.dot(x0, w0, preferred_element_type=jnp.float32)
                + lax.dot(x1, w1, preferred_element_type=jnp.float32)
            ).astype(jnp.bfloat16)
            if c == 0:
                pl.semaphore_wait(barrier_sem, 2)
            d = pltpu.make_async_remote_copy(
                src_ref=send1.at[c], dst_ref=recv1.at[c],
                send_sem=ss1.at[c], recv_sem=rs1.at[c],
                device_id=(pA,), device_id_type=pl.DeviceIdType.MESH,
            )
            d.start()
            rdma1.append(d)

        out_cps = []

        def finish(c):
            rdma2[c].wait_recv()
            outbuf[c, :, :] = send2[c, :, :] + recv2[c, :, :]
            cp = pltpu.make_async_copy(
                outbuf.at[c], out_hbm.at[pl.ds(c * rpc, rpc), :],
                out_sems.at[c],
            )
            cp.start()
            out_cps.append(cp)

        rdma2 = []
        for c in range(NC):
            rdma1[c].wait_recv()
            send2[c, :, :] = send1[c, :, :] + recv1[c, :, :]
            d = pltpu.make_async_remote_copy(
                src_ref=send2.at[c], dst_ref=recv2.at[c],
                send_sem=ss2.at[c], recv_sem=rs2.at[c],
                device_id=(pB,), device_id_type=pl.DeviceIdType.MESH,
            )
            d.start()
            rdma2.append(d)
            if c >= LAG:
                finish(c - LAG)
        for c in range(NC - LAG, NC):
            finish(c)

        for cp in out_cps:
            cp.wait()
        for c in range(NC):
            rdma1[c].wait_send()
            rdma2[c].wait_send()

    return pl.pallas_call(
        body,
        out_shape=jax.ShapeDtypeStruct((n_tok, h), jnp.bfloat16),
        in_specs=[
            pl.BlockSpec(memory_space=pltpu.MemorySpace.HBM),
            pl.BlockSpec(memory_space=pltpu.VMEM),
            pl.BlockSpec(memory_space=pltpu.MemorySpace.HBM),
        ],
        out_specs=pl.BlockSpec(memory_space=pltpu.MemorySpace.HBM),
        scratch_shapes=[
            pltpu.VMEM((n_tok, d), jnp.float32),
            pltpu.VMEM((EXPERTS_PER_DEV, d, h), jnp.float32),
            pltpu.VMEM((NC, rpc, h), jnp.bfloat16),
            pltpu.VMEM((NC, rpc, h), jnp.bfloat16),
            pltpu.VMEM((NC, rpc, h), jnp.bfloat16),
            pltpu.VMEM((NC, rpc, h), jnp.bfloat16),
            pltpu.VMEM((NC, rpc, h), jnp.bfloat16),
            pltpu.SemaphoreType.DMA((2,)),
            pltpu.SemaphoreType.DMA((NC,)),
            pltpu.SemaphoreType.DMA((NC,)),
            pltpu.SemaphoreType.DMA((NC,)),
            pltpu.SemaphoreType.DMA((NC,)),
            pltpu.SemaphoreType.DMA((NC,)),
        ],
        compiler_params=pltpu.CompilerParams(collective_id=0),
    )(
        pltpu.with_memory_space_constraint(x, pltpu.MemorySpace.HBM),
        route_idx,
        pltpu.with_memory_space_constraint(expert_W, pltpu.MemorySpace.HBM),
    )
